# Initial kernel scaffold; baseline (speedup 1.0000x reference)
import jax
import jax.numpy as jnp
from jax.experimental import pallas as pl


def kernel(graph_embedding, edge_index, e_feat, weight):
    raise NotImplementedError("write your pallas kernel here")


# speedup vs baseline: 7.4341x; 7.4341x over previous
import jax
import jax.numpy as jnp
from jax import lax
from jax.experimental import pallas as pl
from jax.experimental.pallas import tpu as pltpu
from jax.experimental.pallas import tpu_sc as plsc

N_NODES = 10000
N_EDGES = 320000
D = 128

NC = 2
NS = 16
C = 128

E_PER_S = N_EDGES // NS
E_PER_S_PAD = 20480
N_CHUNKS = E_PER_S_PAD // C
H = 5120
DUMMY_ROWS = 64
N_ACC = H + DUMMY_ROWS
ROWS_PER_TILE = H // NS


def _elu_body(x_ref, w_ref, o_ref):
    x = x_ref[...] * w_ref[...]
    o_ref[...] = 2.0 * jnp.where(x > 0, x, jnp.exp(jnp.minimum(x, 0.0)) - 1.0)


def _elu_tc(graph_embedding, weight):
    return pl.pallas_call(
        _elu_body,
        out_shape=jax.ShapeDtypeStruct((N_NODES, D), jnp.float32),
    )(graph_embedding, weight)


def _sc_body(emb_hbm, src_hbm, dst_hbm, out_hbm,
             src_v, dst_v, rows0, rows1, acc, gsem):
    cid = lax.axis_index("c")
    sid = lax.axis_index("s")

    pltpu.sync_copy(src_hbm.at[sid], src_v)
    pltpu.sync_copy(dst_hbm.at[cid].at[sid], dst_v)

    zero16 = jnp.zeros((16,), jnp.float32)

    @pl.loop(0, C)
    def _zero_rows(r):
        for c16 in range(D // 16):
            rows0[r, pl.ds(c16 * 16, 16)] = zero16

    row0 = sid * ROWS_PER_TILE
    for k in range(ROWS_PER_TILE // C):
        pltpu.sync_copy(rows0, acc.at[pl.ds(row0 + k * C, C)])
    rem = ROWS_PER_TILE % C
    pltpu.sync_copy(rows0.at[pl.ds(0, rem)],
                    acc.at[pl.ds(row0 + (ROWS_PER_TILE // C) * C, rem)])

    @pl.when(sid == 0)
    def _():
        pltpu.sync_copy(rows0.at[pl.ds(0, DUMMY_ROWS)],
                        acc.at[pl.ds(H, DUMMY_ROWS)])

    plsc.subcore_barrier()

    def start_gather(j, buf):
        pltpu.async_copy(emb_hbm.at[src_v.at[j]], buf, gsem)

    def wait_gather(j, buf):
        pltpu.make_async_copy(emb_hbm.at[src_v.at[j]], buf, gsem).wait()

    def scatter_add(j, buf):
        pltpu.sync_copy(buf, acc.at[dst_v.at[j]], add=True)

    start_gather(0, rows0)

    @pl.loop(0, N_CHUNKS // 2)
    def _pairs(t):
        j0 = 2 * t
        j1 = j0 + 1
        wait_gather(j0, rows0)
        start_gather(j1, rows1)
        scatter_add(j0, rows0)
        wait_gather(j1, rows1)

        @pl.when(j1 + 1 < N_CHUNKS)
        def _():
            start_gather(j1 + 1, rows0)

        scatter_add(j1, rows1)

    plsc.subcore_barrier()

    out_base = cid * H + row0

    @pl.when(out_base + ROWS_PER_TILE <= N_NODES)
    def _():
        pltpu.sync_copy(acc.at[pl.ds(row0, ROWS_PER_TILE)],
                        out_hbm.at[pl.ds(out_base, ROWS_PER_TILE)])

    last = N_NODES - (NC * NS - 1) * ROWS_PER_TILE
    @pl.when(out_base + ROWS_PER_TILE > N_NODES)
    def _():
        pltpu.sync_copy(acc.at[pl.ds(row0, last)],
                        out_hbm.at[pl.ds(out_base, last)])


def _sc_scatter(emb2, src3, dst4):
    mesh = plsc.VectorSubcoreMesh(core_axis_name="c", subcore_axis_name="s",
                                  num_cores=NC, num_subcores=NS)
    return pl.kernel(
        _sc_body,
        out_type=jax.ShapeDtypeStruct((N_NODES, D), jnp.float32),
        mesh=mesh,
        scratch_types=[
            pltpu.VMEM((N_CHUNKS, C), jnp.int32),
            pltpu.VMEM((N_CHUNKS, C), jnp.int32),
            pltpu.VMEM((C, D), jnp.float32),
            pltpu.VMEM((C, D), jnp.float32),
            pltpu.VMEM_SHARED((N_ACC, D), jnp.float32),
            pltpu.SemaphoreType.DMA,
        ],
    )(emb2, src3, dst4)


def kernel(graph_embedding, edge_index, e_feat, weight):
    del e_feat
    src = edge_index[0].astype(jnp.int32).reshape(NS, E_PER_S)
    dst = edge_index[1].astype(jnp.int32).reshape(NS, E_PER_S)

    n_pad = E_PER_S_PAD - E_PER_S
    pad_ar = jnp.arange(NS * n_pad, dtype=jnp.int32).reshape(NS, n_pad)
    pad_src = (pad_ar * 97) % N_NODES
    pad_dst = jnp.full((NS, n_pad), -1, jnp.int32)
    src3 = jnp.concatenate([src, pad_src], axis=1).reshape(NS, N_CHUNKS, C)
    dstp = jnp.concatenate([dst, pad_dst], axis=1)

    rot = jnp.arange(E_PER_S_PAD, dtype=jnp.int32) % DUMMY_ROWS
    dloc = dstp[None] - jnp.array([0, H], jnp.int32)[:, None, None]
    in_range = (dloc >= 0) & (dloc < H)
    dst4 = jnp.where(in_range, dloc, H + rot[None, None, :])
    dst4 = dst4.reshape(NC, NS, N_CHUNKS, C)

    emb2 = _elu_tc(graph_embedding, weight)
    return _sc_scatter(emb2, src3, dst4)


# speedup vs baseline: 10.0412x; 1.3507x over previous
import jax
import jax.numpy as jnp
from jax import lax
from jax.experimental import pallas as pl
from jax.experimental.pallas import tpu as pltpu
from jax.experimental.pallas import tpu_sc as plsc

N_NODES = 10000
N_EDGES = 320000
D = 128

NC = 2
NS = 16
C = 128

E_PER_S = N_EDGES // NS
E_PER_S_PAD = 20480
N_CHUNKS = E_PER_S_PAD // C
H = 5120
DUMMY_ROWS = 64
N_ACC = H + DUMMY_ROWS
ROWS_PER_TILE = H // NS
NBUF = 4
N_PHASES = 2
CHUNKS_PER_PHASE = N_CHUNKS // N_PHASES


def _elu_body(x_ref, w_ref, o_ref):
    x = x_ref[...] * w_ref[...]
    o_ref[...] = 2.0 * jnp.where(x > 0, x, jnp.exp(jnp.minimum(x, 0.0)) - 1.0)


def _elu_tc(graph_embedding, weight):
    return pl.pallas_call(
        _elu_body,
        out_shape=jax.ShapeDtypeStruct((N_NODES, D), jnp.float32),
    )(graph_embedding, weight)


def _sc_body(emb_hbm, src_hbm, dst_hbm, out_hbm,
             src_v, dst_v, rows, acc, gsem):
    cid = lax.axis_index("c")
    sid = lax.axis_index("s")

    zero16 = jnp.zeros((16,), jnp.float32)

    @pl.loop(0, C)
    def _zero_rows(r):
        for c16 in range(D // 16):
            rows[0][r, pl.ds(c16 * 16, 16)] = zero16

    row0 = sid * ROWS_PER_TILE
    for k in range(ROWS_PER_TILE // C):
        pltpu.sync_copy(rows[0], acc.at[pl.ds(row0 + k * C, C)])
    rem = ROWS_PER_TILE % C
    pltpu.sync_copy(rows[0].at[pl.ds(0, rem)],
                    acc.at[pl.ds(row0 + (ROWS_PER_TILE // C) * C, rem)])

    @pl.when(sid == 0)
    def _():
        pltpu.sync_copy(rows[0].at[pl.ds(0, DUMMY_ROWS)],
                        acc.at[pl.ds(H, DUMMY_ROWS)])

    plsc.subcore_barrier()

    base = cid * H
    iota16 = lax.iota(jnp.int32, 16)

    def remap_dst(j):
        for k in range(D // 16):
            v = dst_v[j, pl.ds(k * 16, 16)]
            t = v - base
            ok = (t >= 0) & (t < H)
            r = H + ((iota16 + j * 8 + k) & (DUMMY_ROWS - 1))
            dst_v[j, pl.ds(k * 16, 16)] = jnp.where(ok, t, r)

    def start_gather(j, buf):
        pltpu.async_copy(emb_hbm.at[src_v.at[j]], buf, gsem)

    def wait_gather(j, buf):
        pltpu.make_async_copy(emb_hbm.at[src_v.at[j]], buf, gsem).wait()

    def scatter_add(j, buf):
        pltpu.sync_copy(buf, acc.at[dst_v.at[j]], add=True)

    for p in range(N_PHASES):
        pltpu.sync_copy(src_hbm.at[sid].at[pl.ds(p * CHUNKS_PER_PHASE,
                                                 CHUNKS_PER_PHASE)], src_v)
        pltpu.sync_copy(dst_hbm.at[sid].at[pl.ds(p * CHUNKS_PER_PHASE,
                                                 CHUNKS_PER_PHASE)], dst_v)

        start_gather(0, rows[0])
        start_gather(1, rows[1])
        start_gather(2, rows[2])

        @pl.loop(0, CHUNKS_PER_PHASE // NBUF)
        def _trips(t):
            for b in range(NBUF):
                j = NBUF * t + b
                wait_gather(j, rows[b])
                @pl.when(j + 3 < CHUNKS_PER_PHASE)
                def _():
                    start_gather(j + 3, rows[(b + 3) % NBUF])

                remap_dst(j)
                scatter_add(j, rows[b])

    plsc.subcore_barrier()

    out_base = base + row0

    @pl.when(out_base + ROWS_PER_TILE <= N_NODES)
    def _():
        pltpu.sync_copy(acc.at[pl.ds(row0, ROWS_PER_TILE)],
                        out_hbm.at[pl.ds(out_base, ROWS_PER_TILE)])

    last = N_NODES - (NC * NS - 1) * ROWS_PER_TILE
    @pl.when(out_base + ROWS_PER_TILE > N_NODES)
    def _():
        pltpu.sync_copy(acc.at[pl.ds(row0, last)],
                        out_hbm.at[pl.ds(out_base, last)])


def _sc_scatter(emb2, src3, dst3):
    mesh = plsc.VectorSubcoreMesh(core_axis_name="c", subcore_axis_name="s",
                                  num_cores=NC, num_subcores=NS)
    return pl.kernel(
        _sc_body,
        out_type=jax.ShapeDtypeStruct((N_NODES, D), jnp.float32),
        mesh=mesh,
        scratch_types=[
            pltpu.VMEM((CHUNKS_PER_PHASE, C), jnp.int32),
            pltpu.VMEM((CHUNKS_PER_PHASE, C), jnp.int32),
            [pltpu.VMEM((C, D), jnp.float32)] * NBUF,
            pltpu.VMEM_SHARED((N_ACC, D), jnp.float32),
            pltpu.SemaphoreType.DMA,
        ],
    )(emb2, src3, dst3)


def kernel(graph_embedding, edge_index, e_feat, weight):
    del e_feat
    src = edge_index[0].astype(jnp.int32).reshape(NS, E_PER_S)
    dst = edge_index[1].astype(jnp.int32).reshape(NS, E_PER_S)

    n_pad = E_PER_S_PAD - E_PER_S
    pad_ar = jnp.arange(NS * n_pad, dtype=jnp.int32).reshape(NS, n_pad)
    pad_src = (pad_ar * 97) % N_NODES
    pad_dst = jnp.full((NS, n_pad), -1, jnp.int32)
    src3 = jnp.concatenate([src, pad_src], axis=1).reshape(NS, N_CHUNKS, C)
    dst3 = jnp.concatenate([dst, pad_dst], axis=1).reshape(NS, N_CHUNKS, C)

    emb2 = _elu_tc(graph_embedding, weight)
    return _sc_scatter(emb2, src3, dst3)


# speedup vs baseline: 11.3400x; 1.1294x over previous
import jax
import jax.numpy as jnp
from jax import lax
from jax.experimental import pallas as pl
from jax.experimental.pallas import tpu as pltpu
from jax.experimental.pallas import tpu_sc as plsc

N_NODES = 10000
N_EDGES = 320000
D = 128

NC = 2
NS = 16
NW = NC * NS
C = 128

E_PER_W = N_EDGES // NW
E_PER_W_PAD = 10240
N_CHUNKS = E_PER_W_PAD // C
ROWS_PER_TILE = 632
N_ACC = ROWS_PER_TILE * NS
PAD_ROWS = N_ACC - N_NODES
NBUF = 2
N_PHASES = 5
CHUNKS_PER_PHASE = N_CHUNKS // N_PHASES


def _elu_body(x_ref, w_ref, o_ref):
    x = x_ref[...] * w_ref[...]
    o_ref[...] = 2.0 * jnp.where(x > 0, x, jnp.exp(jnp.minimum(x, 0.0)) - 1.0)


def _elu_tc(graph_embedding, weight):
    return pl.pallas_call(
        _elu_body,
        out_shape=jax.ShapeDtypeStruct((N_NODES, D), jnp.float32),
    )(graph_embedding, weight)


def _combine_body(p_ref, o_ref):
    o_ref[...] = p_ref[0] + p_ref[1]


def _combine_tc(partials):
    return pl.pallas_call(
        _combine_body,
        out_shape=jax.ShapeDtypeStruct((N_NODES, D), jnp.float32),
    )(partials)


def _sc_body(emb_hbm, src_hbm, dst_hbm, out_hbm,
             src_v, dst_v, rows, acc, gsem):
    cid = lax.axis_index("c")
    sid = lax.axis_index("s")

    zero16 = jnp.zeros((16,), jnp.float32)

    @pl.loop(0, C)
    def _zero_rows(r):
        for c16 in range(D // 16):
            rows[0][r, pl.ds(c16 * 16, 16)] = zero16

    row0 = sid * ROWS_PER_TILE
    for k in range(ROWS_PER_TILE // C):
        pltpu.sync_copy(rows[0], acc.at[pl.ds(row0 + k * C, C)])
    rem = ROWS_PER_TILE % C
    pltpu.sync_copy(rows[0].at[pl.ds(0, rem)],
                    acc.at[pl.ds(row0 + (ROWS_PER_TILE // C) * C, rem)])

    plsc.subcore_barrier()

    def start_gather(j, buf):
        pltpu.async_copy(emb_hbm.at[src_v.at[j]], buf, gsem)

    def wait_gather(j, buf):
        pltpu.make_async_copy(emb_hbm.at[src_v.at[j]], buf, gsem).wait()

    def scatter_add(j, buf):
        pltpu.sync_copy(buf, acc.at[dst_v.at[j]], add=True)

    for p in range(N_PHASES):
        pltpu.sync_copy(src_hbm.at[cid].at[sid].at[pl.ds(
            p * CHUNKS_PER_PHASE, CHUNKS_PER_PHASE)], src_v)
        pltpu.sync_copy(dst_hbm.at[cid].at[sid].at[pl.ds(
            p * CHUNKS_PER_PHASE, CHUNKS_PER_PHASE)], dst_v)

        start_gather(0, rows[0])

        @pl.loop(0, CHUNKS_PER_PHASE // NBUF)
        def _trips(t):
            for b in range(NBUF):
                j = NBUF * t + b
                wait_gather(j, rows[b])
                @pl.when(j + 1 < CHUNKS_PER_PHASE)
                def _():
                    start_gather(j + 1, rows[(b + 1) % NBUF])

                scatter_add(j, rows[b])

    plsc.subcore_barrier()

    @pl.when(sid < NS - 1)
    def _():
        pltpu.sync_copy(acc.at[pl.ds(row0, ROWS_PER_TILE)],
                        out_hbm.at[cid].at[pl.ds(row0, ROWS_PER_TILE)])

    @pl.when(sid == NS - 1)
    def _():
        last = N_NODES - (NS - 1) * ROWS_PER_TILE
        pltpu.sync_copy(acc.at[pl.ds(row0, last)],
                        out_hbm.at[cid].at[pl.ds(row0, last)])


def _sc_scatter(emb2, src4, dst4):
    mesh = plsc.VectorSubcoreMesh(core_axis_name="c", subcore_axis_name="s",
                                  num_cores=NC, num_subcores=NS)
    return pl.kernel(
        _sc_body,
        out_type=jax.ShapeDtypeStruct((NC, N_NODES, D), jnp.float32),
        mesh=mesh,
        scratch_types=[
            pltpu.VMEM((CHUNKS_PER_PHASE, C), jnp.int32),
            pltpu.VMEM((CHUNKS_PER_PHASE, C), jnp.int32),
            [pltpu.VMEM((C, D), jnp.float32)] * NBUF,
            pltpu.VMEM_SHARED((N_ACC, D), jnp.float32),
            pltpu.SemaphoreType.DMA,
        ],
    )(emb2, src4, dst4)


def kernel(graph_embedding, edge_index, e_feat, weight):
    del e_feat
    src = edge_index[0].astype(jnp.int32).reshape(NW, E_PER_W)
    dst = edge_index[1].astype(jnp.int32).reshape(NW, E_PER_W)

    n_pad = E_PER_W_PAD - E_PER_W
    pad_ar = jnp.arange(NW * n_pad, dtype=jnp.int32).reshape(NW, n_pad)
    pad_src = (pad_ar * 97) % N_NODES
    pad_dst = N_NODES + (pad_ar % PAD_ROWS)
    src4 = jnp.concatenate([src, pad_src], axis=1).reshape(NC, NS, N_CHUNKS, C)
    dst4 = jnp.concatenate([dst, pad_dst], axis=1).reshape(NC, NS, N_CHUNKS, C)

    emb2 = _elu_tc(graph_embedding, weight)
    partials = _sc_scatter(emb2, src4, dst4)
    return _combine_tc(partials)


# speedup vs baseline: 13.0636x; 1.1520x over previous
import jax
import jax.numpy as jnp
from jax import lax
from jax.experimental import pallas as pl
from jax.experimental.pallas import tpu as pltpu
from jax.experimental.pallas import tpu_sc as plsc

N_NODES = 10000
N_EDGES = 320000
D = 128

NC = 2
NS = 16
NW = NC * NS
C = 120

E_PER_W = N_EDGES // NW
E_PER_W_PAD = 10800
N_CHUNKS = E_PER_W_PAD // C
N_PHASES = 6
K = N_CHUNKS // N_PHASES
NBUF = 3
PAD_PER_W = E_PER_W_PAD - E_PER_W
N_TAB = N_NODES + PAD_PER_W
ROWS_A = 632
ROWS_B = N_NODES - 15 * ROWS_A


def _elu_body(x_ref, w_ref, o_ref):
    x = x_ref[...] * w_ref[...]
    o_ref[pl.ds(0, N_NODES)] = 2.0 * jnp.where(
        x > 0, x, jnp.exp(jnp.minimum(x, 0.0)) - 1.0)
    o_ref[pl.ds(N_NODES, N_TAB - N_NODES)] = jnp.zeros(
        (N_TAB - N_NODES, D), jnp.float32)


def _elu_tc(graph_embedding, weight):
    return pl.pallas_call(
        _elu_body,
        out_shape=jax.ShapeDtypeStruct((N_TAB, D), jnp.float32),
    )(graph_embedding, weight)


def _combine_body(p_ref, o_ref):
    o_ref[...] = p_ref[0] + p_ref[1]


def _combine_tc(partials):
    return pl.pallas_call(
        _combine_body,
        out_shape=jax.ShapeDtypeStruct((N_NODES, D), jnp.float32),
    )(partials)


def _sc_body(emb_hbm, src_hbm, dst_hbm, out_hbm,
             src_v, dst_v, rows, acc, gsem, ssem):
    cid = lax.axis_index("c")
    sid = lax.axis_index("s")

    zero16 = jnp.zeros((16,), jnp.float32)

    @pl.loop(0, C)
    def _zero_rows(r):
        for c16 in range(D // 16):
            rows[0][r, pl.ds(c16 * 16, 16)] = zero16

    row0 = sid * ROWS_A

    def zero_span(n_rows):
        for k in range(n_rows // C):
            pltpu.sync_copy(rows[0], acc.at[pl.ds(row0 + k * C, C)])
        rem = n_rows % C
        pltpu.sync_copy(rows[0].at[pl.ds(0, rem)],
                        acc.at[pl.ds(row0 + (n_rows // C) * C, rem)])

    @pl.when(sid < NS - 1)
    def _():
        zero_span(ROWS_A)

    @pl.when(sid == NS - 1)
    def _():
        zero_span(ROWS_B)

    plsc.subcore_barrier()

    def start_gather(j, buf):
        pltpu.async_copy(emb_hbm.at[src_v.at[j]], buf, gsem)

    def wait_gather(j, buf):
        pltpu.make_async_copy(emb_hbm.at[src_v.at[j]], buf, gsem).wait()

    def start_scatter(j, buf):
        pltpu.async_copy(buf, acc.at[dst_v.at[j]], ssem, add=True)

    def wait_scatter(j, buf):
        pltpu.make_async_copy(buf, acc.at[dst_v.at[j]], ssem).wait()

    for p in range(N_PHASES):
        pltpu.sync_copy(src_hbm.at[cid].at[sid].at[p], src_v)
        pltpu.sync_copy(dst_hbm.at[cid].at[sid].at[p], dst_v)

        start_gather(0, rows[0])
        start_gather(1, rows[1])

        @pl.loop(0, K // NBUF)
        def _trips(t):
            for b in range(NBUF):
                j = NBUF * t + b
                wait_gather(j, rows[b])
                if b == 0:
                    @pl.when(t > 0)
                    def _():
                        wait_scatter(j - 1, rows[(b + 2) % NBUF])
                    start_gather(j + 2, rows[(b + 2) % NBUF])
                else:
                    wait_scatter(j - 1, rows[(b + 2) % NBUF])

                    @pl.when(t < K // NBUF - 1)
                    def _():
                        start_gather(j + 2, rows[(b + 2) % NBUF])

                start_scatter(j, rows[b])

        wait_scatter(K - 1, rows[(K - 1) % NBUF])

    plsc.subcore_barrier()

    @pl.when(sid < NS - 1)
    def _():
        pltpu.sync_copy(acc.at[pl.ds(row0, ROWS_A)],
                        out_hbm.at[cid].at[pl.ds(row0, ROWS_A)])

    @pl.when(sid == NS - 1)
    def _():
        pltpu.sync_copy(acc.at[pl.ds(row0, ROWS_B)],
                        out_hbm.at[cid].at[pl.ds(row0, ROWS_B)])


def _sc_scatter(emb2, src5, dst5):
    mesh = plsc.VectorSubcoreMesh(core_axis_name="c", subcore_axis_name="s",
                                  num_cores=NC, num_subcores=NS)
    return pl.kernel(
        _sc_body,
        out_type=jax.ShapeDtypeStruct((NC, N_NODES, D), jnp.float32),
        mesh=mesh,
        scratch_types=[
            pltpu.VMEM((K, C), jnp.int32),
            pltpu.VMEM((K, C), jnp.int32),
            [pltpu.VMEM((C, D), jnp.float32)] * NBUF,
            pltpu.VMEM_SHARED((N_NODES, D), jnp.float32),
            pltpu.SemaphoreType.DMA,
            pltpu.SemaphoreType.DMA,
        ],
    )(emb2, src5, dst5)


def kernel(graph_embedding, edge_index, e_feat, weight):
    del e_feat
    src = edge_index[0].astype(jnp.int32).reshape(NW, E_PER_W)
    dst = edge_index[1].astype(jnp.int32).reshape(NW, E_PER_W)

    pad_ar = jnp.arange(NW * PAD_PER_W, dtype=jnp.int32).reshape(NW, PAD_PER_W)
    pad_src = N_NODES + (pad_ar % PAD_PER_W)
    pad_dst = (pad_ar * 97) % N_NODES
    src5 = jnp.concatenate([src, pad_src], axis=1).reshape(
        NC, NS, N_PHASES, K, C)
    dst5 = jnp.concatenate([dst, pad_dst], axis=1).reshape(
        NC, NS, N_PHASES, K, C)

    emb2 = _elu_tc(graph_embedding, weight)
    partials = _sc_scatter(emb2, src5, dst5)
    return _combine_tc(partials)


# speedup vs baseline: 15.0290x; 1.1504x over previous
import jax
import jax.numpy as jnp
from jax import lax
from jax.experimental import pallas as pl
from jax.experimental.pallas import tpu as pltpu
from jax.experimental.pallas import tpu_sc as plsc

N_NODES = 10000
N_EDGES = 320000
D = 128

NC = 2
NS = 16
NW = NC * NS
C = 112

E_PER_W = N_EDGES // NW
N_FULL = E_PER_W // C
TAIL = E_PER_W - N_FULL * C
K = 15
N_PH_FULL = 5
K_LAST = N_FULL - N_PH_FULL * K
NBUF = 3
ROWS_A = 632
ROWS_B = N_NODES - 15 * ROWS_A
IDX_WORDS = K * C


def _elu_body(x_ref, w_ref, o_ref):
    x = x_ref[...] * w_ref[...]
    o_ref[...] = 2.0 * jnp.where(x > 0, x, jnp.exp(jnp.minimum(x, 0.0)) - 1.0)


def _elu_tc(graph_embedding, weight):
    return pl.pallas_call(
        _elu_body,
        out_shape=jax.ShapeDtypeStruct((N_NODES, D), jnp.float32),
    )(graph_embedding, weight)


def _combine_body(p_ref, o_ref):
    o_ref[...] = p_ref[0] + p_ref[1]


def _combine_tc(partials):
    return pl.pallas_call(
        _combine_body,
        out_shape=jax.ShapeDtypeStruct((N_NODES, D), jnp.float32),
    )(partials)


def _sc_body(emb_hbm, idx_hbm, out_hbm,
             src_v, dst1d_v, dst_v, dstt_v, rows, acc, gsem, ssem):
    cid = lax.axis_index("c")
    sid = lax.axis_index("s")
    base = (cid * NS + sid) * E_PER_W

    zero16 = jnp.zeros((16,), jnp.float32)

    @pl.loop(0, C)
    def _zero_rows(r):
        for c16 in range(D // 16):
            rows[0][r, pl.ds(c16 * 16, 16)] = zero16

    row0 = sid * ROWS_A

    def zero_span(n_rows):
        for k in range(n_rows // C):
            pltpu.sync_copy(rows[0], acc.at[pl.ds(row0 + k * C, C)])
        rem = n_rows % C
        pltpu.sync_copy(rows[0].at[pl.ds(0, rem)],
                        acc.at[pl.ds(row0 + (n_rows // C) * C, rem)])

    @pl.when(sid < NS - 1)
    def _():
        zero_span(ROWS_A)

    @pl.when(sid == NS - 1)
    def _():
        zero_span(ROWS_B)

    plsc.subcore_barrier()

    def start_gather(j, buf):
        pltpu.async_copy(emb_hbm.at[src_v.at[pl.ds(j * C, C)]], buf, gsem)

    def wait_gather(j, buf):
        pltpu.make_async_copy(
            emb_hbm.at[src_v.at[pl.ds(j * C, C)]], buf, gsem).wait()

    def start_scatter(j, buf):
        pltpu.async_copy(buf, acc.at[dst_v.at[j]], ssem, add=True)

    def wait_scatter(j, buf):
        pltpu.make_async_copy(buf, acc.at[dst_v.at[j]], ssem).wait()

    def repack_dst(j):
        for k in range(C // 16):
            dst_v[j, pl.ds(k * 16, 16)] = dst1d_v[pl.ds(j * C + k * 16, 16)]

    def chunk_body(j, b, n):
        wait_gather(j, rows[b])

        @pl.when(j >= 1)
        def _():
            wait_scatter(j - 1, rows[(b + 2) % NBUF])

        @pl.when(j + 2 < n)
        def _():
            start_gather(j + 2, rows[(b + 2) % NBUF])

        repack_dst(j)
        start_scatter(j, rows[b])

    def run_phase(p, n):
        words = n * C
        pltpu.sync_copy(idx_hbm.at[pl.ds(base + p * IDX_WORDS, words)],
                        src_v.at[pl.ds(0, words)])
        pltpu.sync_copy(idx_hbm.at[pl.ds(N_EDGES + base + p * IDX_WORDS,
                                         words)],
                        dst1d_v.at[pl.ds(0, words)])
        start_gather(0, rows[0])
        start_gather(1, rows[1])

        @pl.loop(0, n // NBUF)
        def _trips(t):
            for b in range(NBUF):
                chunk_body(NBUF * t + b, b, n)

        for j in range((n // NBUF) * NBUF, n):
            chunk_body(j, j % NBUF, n)

        wait_scatter(n - 1, rows[(n - 1) % NBUF])

    for p in range(N_PH_FULL):
        run_phase(p, K)
    run_phase(N_PH_FULL, K_LAST)

    toff = N_PH_FULL * IDX_WORDS + K_LAST * C
    pltpu.sync_copy(idx_hbm.at[pl.ds(base + toff, TAIL)],
                    src_v.at[pl.ds(0, TAIL)])
    pltpu.sync_copy(idx_hbm.at[pl.ds(N_EDGES + base + toff, TAIL)],
                    dst1d_v.at[pl.ds(0, TAIL)])
    for k in range(TAIL // 16):
        dstt_v[0, pl.ds(k * 16, 16)] = dst1d_v[pl.ds(k * 16, 16)]
    pltpu.async_copy(emb_hbm.at[src_v.at[pl.ds(0, TAIL)]],
                     rows[0].at[pl.ds(0, TAIL)], gsem)
    pltpu.make_async_copy(emb_hbm.at[src_v.at[pl.ds(0, TAIL)]],
                          rows[0].at[pl.ds(0, TAIL)], gsem).wait()
    pltpu.sync_copy(rows[0].at[pl.ds(0, TAIL)],
                    acc.at[dstt_v.at[0]], add=True)

    plsc.subcore_barrier()

    @pl.when(sid < NS - 1)
    def _():
        pltpu.sync_copy(acc.at[pl.ds(row0, ROWS_A)],
                        out_hbm.at[cid].at[pl.ds(row0, ROWS_A)])

    @pl.when(sid == NS - 1)
    def _():
        pltpu.sync_copy(acc.at[pl.ds(row0, ROWS_B)],
                        out_hbm.at[cid].at[pl.ds(row0, ROWS_B)])


def _sc_scatter(emb2, idx_flat):
    mesh = plsc.VectorSubcoreMesh(core_axis_name="c", subcore_axis_name="s",
                                  num_cores=NC, num_subcores=NS)
    return pl.kernel(
        _sc_body,
        out_type=jax.ShapeDtypeStruct((NC, N_NODES, D), jnp.float32),
        mesh=mesh,
        scratch_types=[
            pltpu.VMEM((IDX_WORDS,), jnp.int32),
            pltpu.VMEM((IDX_WORDS,), jnp.int32),
            pltpu.VMEM((K, C), jnp.int32),
            pltpu.VMEM((8, TAIL), jnp.int32),
            [pltpu.VMEM((C, D), jnp.float32)] * NBUF,
            pltpu.VMEM_SHARED((N_NODES, D), jnp.float32),
            pltpu.SemaphoreType.DMA,
            pltpu.SemaphoreType.DMA,
        ],
    )(emb2, idx_flat)


def kernel(graph_embedding, edge_index, e_feat, weight):
    del e_feat
    idx_flat = edge_index.astype(jnp.int32).reshape(-1)
    emb2 = _elu_tc(graph_embedding, weight)
    partials = _sc_scatter(emb2, idx_flat)
    return _combine_tc(partials)
